# baseline (speedup 1.0000x reference)
import jax
import jax.numpy as jnp
from jax.experimental import pallas as pl

_INV_COEFF = 25.0
_VAR_COEFF = 25.0
_COV_COEFF = 1.0


def _first_argmin_onehot(D, axis):
    N, M = D.shape
    iota = jax.lax.broadcasted_iota(jnp.int32, (N, M), axis)
    mn = jnp.min(D, axis=axis, keepdims=True)
    big = jnp.int32(D.shape[axis])
    cand = jnp.where(D == mn, iota, big)
    idx = jnp.min(cand, axis=axis, keepdims=True)
    return (iota == idx).astype(jnp.bfloat16), mn


def _dot_t(a, b):
    return jax.lax.dot_general(
        a, b, (((1,), (1,)), ((), ())),
        preferred_element_type=jnp.float32)


def _dot(a, b):
    return jax.lax.dot_general(
        a, b, (((1,), (0,)), ((), ())),
        preferred_element_type=jnp.float32)


def _dot_lt(a, b):
    return jax.lax.dot_general(
        a, b, (((0,), (0,)), ((), ())),
        preferred_element_type=jnp.float32)


def _side_stats(x):
    B, N, C = x.shape
    mu = jnp.mean(x, axis=0)
    sq = x * x
    szz = jnp.sum(sq, axis=0) - B * (mu * mu)
    std = jnp.sqrt(szz / (B - 1) + 0.0001)
    var_term = jnp.mean(jnp.maximum(1.0 - std, 0.0))

    ts = {}
    g2 = jnp.zeros((N,), dtype=jnp.float32)
    for p in range(B):
        tpp = jnp.sum(sq[p], axis=-1)
        ts[(p, p)] = tpp
        g2 = g2 + tpp * tpp
        for q in range(p + 1, B):
            t = jnp.sum(x[p] * x[q], axis=-1)
            ts[(p, q)] = t
            g2 = g2 + 2.0 * (t * t)
    su2 = jnp.zeros((N,), dtype=jnp.float32)
    stot = jnp.zeros((N,), dtype=jnp.float32)
    for p in range(B):
        s_p = ts[(p, p)]
        for q in range(B):
            if q != p:
                s_p = s_p + ts[(min(p, q), max(p, q))]
        su2 = su2 + (s_p / B) * (s_p / B)
        stot = stot + s_p
    m = stot / (B * B)
    gc2 = g2 - (2.0 * B) * su2 + float(B * B) * (m * m)
    diag = jnp.sum(szz * szz, axis=-1)
    off = (gc2 - diag) / float((B - 1) * (B - 1))
    return var_term, off


def _vicreg_terms(x, y):
    B, N, C = x.shape
    vx, off_x = _side_stats(x)
    vy, off_y = _side_stats(y)
    var = _VAR_COEFF * (vx / 2 + vy / 2)
    cov = _COV_COEFF * jnp.mean(off_x / C / 2 + off_y / C / 2)
    return var, cov


def _loss_kernel(m1_ref, m2_ref, out_ref):
    a = m1_ref[...]
    b = m2_ref[...]
    B, N, C = a.shape
    a2 = jnp.sum(a * a, axis=-1)
    b2 = jnp.sum(b * b, axis=-1)

    ab16 = a.astype(jnp.bfloat16)
    bb16 = b.astype(jnp.bfloat16)
    n1_rows = []
    n2_rows = []
    min_sum = jnp.zeros((), dtype=jnp.float32)
    for i in range(B):
        A = ab16[i]
        Bm = bb16[i]
        D = a2[i][:, None] + b2[i][None, :] - 2.0 * _dot_t(A, Bm)
        oh1, mn1 = _first_argmin_onehot(D, axis=1)
        oh2, mn2 = _first_argmin_onehot(D, axis=0)
        min_sum = min_sum + jnp.sum(mn1) + jnp.sum(mn2)
        n1_rows.append(_dot(oh1, Bm))
        n2_rows.append(_dot_lt(oh2, A))
    n1 = jnp.stack(n1_rows)
    n2 = jnp.stack(n2_rows)

    inv = _INV_COEFF * min_sum / (2.0 * B * N * C)

    v1, c1 = _vicreg_terms(a, n1)
    v2, c2 = _vicreg_terms(b, n2)
    var = v1 / 2 + v2 / 2
    cov = c1 / 2 + c2 / 2

    lane = jax.lax.broadcasted_iota(jnp.int32, (1, 128), 1)
    vals = jnp.where(lane == 0, inv, jnp.where(lane == 1, var, cov))
    out_ref[...] = vals


def kernel(maps_1, maps_2):
    B, V, C, H, W = maps_1.shape
    m1 = jnp.transpose(maps_1.reshape(B * V, C, H * W), (0, 2, 1))
    m2 = jnp.transpose(maps_2.reshape(B * V, C, H * W), (0, 2, 1))
    out = pl.pallas_call(
        _loss_kernel,
        out_shape=jax.ShapeDtypeStruct((1, 128), jnp.float32),
    )(m1, m2)
    return out[0, :3]


# speedup vs baseline: 1.0137x; 1.0137x over previous
import jax
import jax.numpy as jnp
from jax.experimental import pallas as pl

_INV_COEFF = 25.0
_VAR_COEFF = 25.0
_COV_COEFF = 1.0


def _first_argmin_onehot(D, axis):
    N, M = D.shape
    iota = jax.lax.broadcasted_iota(jnp.int32, (N, M), axis)
    mn = jnp.min(D, axis=axis, keepdims=True)
    big = jnp.int32(D.shape[axis])
    cand = jnp.where(D == mn, iota, big)
    idx = jnp.min(cand, axis=axis, keepdims=True)
    return (iota == idx).astype(jnp.bfloat16), mn


def _dot_t(a, b):
    return jax.lax.dot_general(
        a, b, (((1,), (1,)), ((), ())),
        preferred_element_type=jnp.float32)


def _dot(a, b):
    return jax.lax.dot_general(
        a, b, (((1,), (0,)), ((), ())),
        preferred_element_type=jnp.float32)


def _dot_lt(a, b):
    return jax.lax.dot_general(
        a, b, (((0,), (0,)), ((), ())),
        preferred_element_type=jnp.float32)


def _side_stats(x, tpp=None):
    B, N, C = x.shape
    mu = jnp.mean(x, axis=0)
    sq = x * x
    szz = jnp.sum(sq, axis=0) - B * (mu * mu)
    std = jnp.sqrt(szz / (B - 1) + 0.0001)
    var_term = jnp.mean(jnp.maximum(1.0 - std, 0.0))

    g2 = jnp.zeros((N,), dtype=jnp.float32)
    s = [jnp.zeros((N,), dtype=jnp.float32) for _ in range(B)]
    for p in range(B):
        t_pp = jnp.sum(sq[p], axis=-1) if tpp is None else tpp[p]
        g2 = g2 + t_pp * t_pp
        s[p] = s[p] + t_pp
        for q in range(p + 1, B):
            t = jnp.sum(x[p] * x[q], axis=-1)
            g2 = g2 + 2.0 * (t * t)
            s[p] = s[p] + t
            s[q] = s[q] + t
    su2 = jnp.zeros((N,), dtype=jnp.float32)
    stot = jnp.zeros((N,), dtype=jnp.float32)
    for p in range(B):
        su2 = su2 + (s[p] / B) * (s[p] / B)
        stot = stot + s[p]
    m = stot / (B * B)
    gc2 = g2 - (2.0 * B) * su2 + float(B * B) * (m * m)
    diag = jnp.sum(szz * szz, axis=-1)
    off = (gc2 - diag) / float((B - 1) * (B - 1))
    return var_term, off


def _vicreg_terms(x, y, x_tpp):
    B, N, C = x.shape
    vx, off_x = _side_stats(x, x_tpp)
    vy, off_y = _side_stats(y)
    var = _VAR_COEFF * (vx / 2 + vy / 2)
    cov = _COV_COEFF * jnp.mean(off_x / C / 2 + off_y / C / 2)
    return var, cov


def _loss_kernel(m1_ref, m2_ref, out_ref):
    a = m1_ref[...]
    b = m2_ref[...]
    B, N, C = a.shape
    a2 = jnp.sum(a * a, axis=-1)
    b2 = jnp.sum(b * b, axis=-1)

    ab16 = a.astype(jnp.bfloat16)
    bb16 = b.astype(jnp.bfloat16)
    n1_rows = []
    n2_rows = []
    min_sum = jnp.zeros((), dtype=jnp.float32)
    for i in range(B):
        A = ab16[i]
        Bm = bb16[i]
        D = a2[i][:, None] + b2[i][None, :] - 2.0 * _dot_t(A, Bm)
        oh1, mn1 = _first_argmin_onehot(D, axis=1)
        oh2, mn2 = _first_argmin_onehot(D, axis=0)
        min_sum = min_sum + jnp.sum(mn1) + jnp.sum(mn2)
        n1_rows.append(_dot(oh1, Bm))
        n2_rows.append(_dot_lt(oh2, A))
    n1 = jnp.stack(n1_rows)
    n2 = jnp.stack(n2_rows)

    inv = _INV_COEFF * min_sum / (2.0 * B * N * C)

    v1, c1 = _vicreg_terms(a, n1, a2)
    v2, c2 = _vicreg_terms(b, n2, b2)
    var = v1 / 2 + v2 / 2
    cov = c1 / 2 + c2 / 2

    lane = jax.lax.broadcasted_iota(jnp.int32, (1, 128), 1)
    vals = jnp.where(lane == 0, inv, jnp.where(lane == 1, var, cov))
    out_ref[...] = vals


def kernel(maps_1, maps_2):
    B, V, C, H, W = maps_1.shape
    m1 = jnp.transpose(maps_1.reshape(B * V, C, H * W), (0, 2, 1))
    m2 = jnp.transpose(maps_2.reshape(B * V, C, H * W), (0, 2, 1))
    out = pl.pallas_call(
        _loss_kernel,
        out_shape=jax.ShapeDtypeStruct((1, 128), jnp.float32),
    )(m1, m2)
    return out[0, :3]


# speedup vs baseline: 1.0853x; 1.0707x over previous
import jax
import jax.numpy as jnp
from jax.experimental import pallas as pl

_INV_COEFF = 25.0
_VAR_COEFF = 25.0
_COV_COEFF = 1.0


def _first_argmin_onehot(D, axis):
    N, M = D.shape
    iota = jax.lax.broadcasted_iota(jnp.int32, (N, M), axis)
    mn = jnp.min(D, axis=axis, keepdims=True)
    big = jnp.int32(D.shape[axis])
    cand = jnp.where(D == mn, iota, big)
    idx = jnp.min(cand, axis=axis, keepdims=True)
    return (iota == idx).astype(jnp.bfloat16), mn


def _dot_t(a, b):
    return jax.lax.dot_general(
        a, b, (((1,), (1,)), ((), ())),
        preferred_element_type=jnp.float32)


def _dot(a, b):
    return jax.lax.dot_general(
        a, b, (((1,), (0,)), ((), ())),
        preferred_element_type=jnp.float32)


def _dot_lt(a, b):
    return jax.lax.dot_general(
        a, b, (((0,), (0,)), ((), ())),
        preferred_element_type=jnp.float32)


def _vicreg_terms(x, y):
    B, N, C = x.shape

    xc = x - jnp.mean(x, axis=0)
    yc = y - jnp.mean(y, axis=0)

    def side_stats(z):
        sq = z * z
        szz = jnp.sum(sq, axis=0)
        std = jnp.sqrt(szz / (B - 1) + 0.0001)
        var_term = jnp.mean(jnp.maximum(1.0 - std, 0.0))
        acc = jnp.zeros((N,), dtype=jnp.float32)
        for p in range(B):
            t = jnp.sum(sq[p], axis=-1)
            acc = acc + t * t
            for q in range(p + 1, B):
                t = jnp.sum(z[p] * z[q], axis=-1)
                acc = acc + 2.0 * (t * t)
        diag = jnp.sum(szz * szz, axis=-1)
        off = (acc - diag) / float((B - 1) * (B - 1))
        return var_term, off

    vx, off_x = side_stats(xc)
    vy, off_y = side_stats(yc)
    var = _VAR_COEFF * (vx / 2 + vy / 2)
    cov = _COV_COEFF * jnp.mean(off_x / C / 2 + off_y / C / 2)
    return var, cov


def _loss_kernel(m1_ref, m2_ref, out_ref):
    a = m1_ref[...]
    b = m2_ref[...]
    B, N, C = a.shape
    a2 = jnp.sum(a * a, axis=-1)
    b2 = jnp.sum(b * b, axis=-1)

    ab16 = a.astype(jnp.bfloat16)
    bb16 = b.astype(jnp.bfloat16)
    n1_rows = []
    n2_rows = []
    min_sum = jnp.zeros((), dtype=jnp.float32)
    for i in range(B):
        A = ab16[i]
        Bm = bb16[i]
        D = a2[i][:, None] + b2[i][None, :] - 2.0 * _dot_t(A, Bm)
        oh1, mn1 = _first_argmin_onehot(D, axis=1)
        oh2, mn2 = _first_argmin_onehot(D, axis=0)
        min_sum = min_sum + jnp.sum(mn1) + jnp.sum(mn2)
        n1_rows.append(_dot(oh1, Bm))
        n2_rows.append(_dot_lt(oh2, A))
    n1 = jnp.stack(n1_rows)
    n2 = jnp.stack(n2_rows)

    inv = _INV_COEFF * min_sum / (2.0 * B * N * C)

    v1, c1 = _vicreg_terms(a, n1)
    v2, c2 = _vicreg_terms(b, n2)
    var = v1 / 2 + v2 / 2
    cov = c1 / 2 + c2 / 2

    lane = jax.lax.broadcasted_iota(jnp.int32, (1, 128), 1)
    vals = jnp.where(lane == 0, inv, jnp.where(lane == 1, var, cov))
    out_ref[...] = vals


def kernel(maps_1, maps_2):
    B, V, C, H, W = maps_1.shape
    m1 = jnp.transpose(maps_1.reshape(B * V, C, H * W), (0, 2, 1))
    m2 = jnp.transpose(maps_2.reshape(B * V, C, H * W), (0, 2, 1))
    out = pl.pallas_call(
        _loss_kernel,
        out_shape=jax.ShapeDtypeStruct((1, 128), jnp.float32),
    )(m1, m2)
    return out[0, :3]


# speedup vs baseline: 1.0903x; 1.0046x over previous
import jax
import jax.numpy as jnp
from jax.experimental import pallas as pl
from jax.experimental.pallas import tpu as pltpu

_INV_COEFF = 25.0
_VAR_COEFF = 25.0
_COV_COEFF = 1.0


def _first_argmin_onehot(D, axis):
    N, M = D.shape
    iota = jax.lax.broadcasted_iota(jnp.int32, (N, M), axis)
    mn = jnp.min(D, axis=axis, keepdims=True)
    big = jnp.int32(D.shape[axis])
    cand = jnp.where(D == mn, iota, big)
    idx = jnp.min(cand, axis=axis, keepdims=True)
    return (iota == idx).astype(jnp.bfloat16), mn


def _dot_t(a, b):
    return jax.lax.dot_general(
        a, b, (((1,), (1,)), ((), ())),
        preferred_element_type=jnp.float32)


def _dot(a, b):
    return jax.lax.dot_general(
        a, b, (((1,), (0,)), ((), ())),
        preferred_element_type=jnp.float32)


def _dot_lt(a, b):
    return jax.lax.dot_general(
        a, b, (((0,), (0,)), ((), ())),
        preferred_element_type=jnp.float32)


def _vicreg_terms(x, y):
    B, N, C = x.shape

    xc = x - jnp.mean(x, axis=0)
    yc = y - jnp.mean(y, axis=0)

    def side_stats(z):
        sq = z * z
        szz = jnp.sum(sq, axis=0)
        std = jnp.sqrt(szz / (B - 1) + 0.0001)
        var_term = jnp.mean(jnp.maximum(1.0 - std, 0.0))
        acc = jnp.zeros((N,), dtype=jnp.float32)
        for p in range(B):
            t = jnp.sum(sq[p], axis=-1)
            acc = acc + t * t
            for q in range(p + 1, B):
                t = jnp.sum(z[p] * z[q], axis=-1)
                acc = acc + 2.0 * (t * t)
        diag = jnp.sum(szz * szz, axis=-1)
        off = (acc - diag) / float((B - 1) * (B - 1))
        return var_term, off

    vx, off_x = side_stats(xc)
    vy, off_y = side_stats(yc)
    var = _VAR_COEFF * (vx / 2 + vy / 2)
    cov = _COV_COEFF * jnp.mean(off_x / C / 2 + off_y / C / 2)
    return var, cov


def _loss_kernel(m1_ref, m2_ref, out_ref, n1_ref, n2_ref):
    a = m1_ref[...]
    b = m2_ref[...]
    B, N, C = a.shape
    a2 = jnp.sum(a * a, axis=-1)
    b2 = jnp.sum(b * b, axis=-1)

    ab16 = a.astype(jnp.bfloat16)
    bb16 = b.astype(jnp.bfloat16)
    min_sum = jnp.zeros((), dtype=jnp.float32)
    for i in range(B):
        A = ab16[i]
        Bm = bb16[i]
        D = a2[i][:, None] + b2[i][None, :] - 2.0 * _dot_t(A, Bm)
        oh1, mn1 = _first_argmin_onehot(D, axis=1)
        oh2, mn2 = _first_argmin_onehot(D, axis=0)
        min_sum = min_sum + jnp.sum(mn1) + jnp.sum(mn2)
        n1_ref[i] = _dot(oh1, Bm)
        n2_ref[i] = _dot_lt(oh2, A)
    n1 = n1_ref[...]
    n2 = n2_ref[...]

    inv = _INV_COEFF * min_sum / (2.0 * B * N * C)

    v1, c1 = _vicreg_terms(a, n1)
    v2, c2 = _vicreg_terms(b, n2)
    var = v1 / 2 + v2 / 2
    cov = c1 / 2 + c2 / 2

    lane = jax.lax.broadcasted_iota(jnp.int32, (1, 128), 1)
    vals = jnp.where(lane == 0, inv, jnp.where(lane == 1, var, cov))
    out_ref[...] = vals


def kernel(maps_1, maps_2):
    B, V, C, H, W = maps_1.shape
    m1 = jnp.transpose(maps_1.reshape(B * V, C, H * W), (0, 2, 1))
    m2 = jnp.transpose(maps_2.reshape(B * V, C, H * W), (0, 2, 1))
    out = pl.pallas_call(
        _loss_kernel,
        out_shape=jax.ShapeDtypeStruct((1, 128), jnp.float32),
        scratch_shapes=[
            pltpu.VMEM((B * V, H * W, C), jnp.float32),
            pltpu.VMEM((B * V, H * W, C), jnp.float32),
        ],
    )(m1, m2)
    return out[0, :3]
